# baseline (speedup 1.0000x reference)
import functools

import jax
import jax.numpy as jnp
from jax import lax
from jax.experimental import pallas as pl
from jax.experimental.pallas import tpu as pltpu
from jax.experimental.pallas import tpu_sc as plsc

N = 10000
E = 320000
H = 128
HH = H // 2
EPS = 1e-6

NC = 2
NS = 16
CHUNK = 64
NBUF = 4
NPAD = 10240
ROWS_PER_SUB = NPAD // NS
CPW = 320
STG = CPW // 8
NCH = NS * CPW
EPAD = NCH * CHUNK
MSTAGE = 624


def _mm_body(x_ref, wt_ref, b_ref, o_ref):
    acc = jnp.dot(x_ref[...], wt_ref[...], preferred_element_type=jnp.float32)
    m = jnp.maximum(acc + b_ref[...], 0.0)
    o_ref[0] = m[:, :HH]
    o_ref[1] = m[:, HH:]


def _linear_relu(x, wt, b2):
    blk = 1000
    return pl.pallas_call(
        _mm_body,
        grid=(N // blk,),
        in_specs=[
            pl.BlockSpec((blk, H), lambda i: (i, 0)),
            pl.BlockSpec((H, H), lambda i: (0, 0)),
            pl.BlockSpec((1, H), lambda i: (0, 0)),
        ],
        out_specs=pl.BlockSpec((2, blk, HH), lambda i: (0, i, 0)),
        out_shape=jax.ShapeDtypeStruct((2, N, HH), jnp.float32),
    )(x, wt, b2)


def _sc_body(m_hbm, ei_hbm, zacc_hbm, zdeg_hbm, ones_hbm,
             agg_out, deg_out,
             row_v, col_v, b0, b1, b2, b3, ones_v, m_sp, acc_s, deg_s,
             g0, g1, g2, g3, s0, s1, s2, s3, sem_d):
    c = lax.axis_index("c")
    s = lax.axis_index("s")

    pltpu.sync_copy(ones_hbm, ones_v)

    pltpu.sync_copy(m_hbm.at[c, pl.ds(s * MSTAGE, MSTAGE)],
                    m_sp.at[pl.ds(s * MSTAGE, MSTAGE)])

    @pl.when(s == NS - 1)
    def _():
        pltpu.sync_copy(m_hbm.at[c, pl.ds(NS * MSTAGE, N - NS * MSTAGE)],
                        m_sp.at[pl.ds(NS * MSTAGE, N - NS * MSTAGE)])

    r0 = s * ROWS_PER_SUB
    pltpu.sync_copy(zacc_hbm, acc_s.at[pl.ds(r0, ROWS_PER_SUB)])
    pltpu.sync_copy(zdeg_hbm, deg_s.at[pl.ds(r0, ROWS_PER_SUB)])
    plsc.subcore_barrier()

    bufs = (b0, b1, b2, b3)
    gsems = (g0, g1, g2, g3)
    ssems = (s0, s1, s2, s3)

    def gather_issue(lc, t):
        idx = col_v.at[pl.ds(lc * CHUNK, CHUNK)]
        hbm_sel = lax.rem(lc, 8) == 7

        @pl.when(hbm_sel)
        def _():
            pltpu.async_copy(m_hbm.at[c].at[idx], bufs[t], gsems[t])

        @pl.when(jnp.logical_not(hbm_sel))
        def _():
            pltpu.async_copy(m_sp.at[idx], bufs[t], gsems[t])

    def gather_wait(lc, t):
        pltpu.make_async_copy(m_sp.at[col_v.at[pl.ds(lc * CHUNK, CHUNK)]], bufs[t], gsems[t]).wait()


    def scatter_wait(lc, t):
        pltpu.make_async_copy(bufs[t], acc_s.at[row_v.at[pl.ds(lc * CHUNK, CHUNK)]],
                              ssems[t]).wait()

    for q in range(8):
        deg_core = 0 if q < 4 else 1
        base = (s * CPW + q * STG) * CHUNK
        pltpu.sync_copy(ei_hbm.at[0, pl.ds(base, STG * CHUNK)], row_v)
        pltpu.sync_copy(ei_hbm.at[1, pl.ds(base, STG * CHUNK)], col_v)

        gather_issue(0, 0)
        gather_issue(1, 1)

        def body(j, carry):
            for t in range(NBUF):
                lc = j * NBUF + t
                gather_wait(lc, t)
                pltpu.async_copy(bufs[t], acc_s.at[row_v.at[pl.ds(lc * CHUNK, CHUNK)]],
                                 ssems[t], add=True)

                @pl.when(c == deg_core)
                def _():
                    pltpu.async_copy(ones_v, deg_s.at[row_v.at[pl.ds(lc * CHUNK, CHUNK)]],
                                     sem_d, add=True)
                tp = (t + 2) % NBUF

                @pl.when(lc + 2 < STG)
                def _():
                    @pl.when(lc >= 2)
                    def _():
                        scatter_wait(lc - 2, tp)
                    gather_issue(lc + 2, tp)
            return carry

        lax.fori_loop(0, STG // NBUF, body, 0)

        for t in range(NBUF):
            scatter_wait(STG - NBUF + t, t)

        def deg_drain(i, carry):
            pltpu.make_async_copy(ones_v, deg_s.at[row_v.at[pl.ds(i * CHUNK, CHUNK)]],
                                  sem_d).wait()
            return carry

        @pl.when(c == deg_core)
        def _():
            lax.fori_loop(0, STG, deg_drain, 0)

    plsc.subcore_barrier()
    pltpu.sync_copy(acc_s.at[pl.ds(r0, ROWS_PER_SUB)],
                    agg_out.at[c, pl.ds(r0, ROWS_PER_SUB)])

    pltpu.sync_copy(deg_s.at[pl.ds(r0, ROWS_PER_SUB)],
                    deg_out.at[pl.ds(c * NPAD + r0, ROWS_PER_SUB)])


_sc_aggregate = functools.partial(
    pl.kernel,
    out_type=(
        jax.ShapeDtypeStruct((NC, NPAD, HH), jnp.float32),
        jax.ShapeDtypeStruct((NC * NPAD,), jnp.float32),
    ),
    mesh=plsc.VectorSubcoreMesh(core_axis_name="c", subcore_axis_name="s"),
    compiler_params=pltpu.CompilerParams(use_tc_tiling_on_sc=False),
    scratch_types=[
        pltpu.VMEM((STG * CHUNK,), jnp.int32),
        pltpu.VMEM((STG * CHUNK,), jnp.int32),
        pltpu.VMEM((CHUNK, HH), jnp.float32),
        pltpu.VMEM((CHUNK, HH), jnp.float32),
        pltpu.VMEM((CHUNK, HH), jnp.float32),
        pltpu.VMEM((CHUNK, HH), jnp.float32),
        pltpu.VMEM((CHUNK,), jnp.float32),
        pltpu.VMEM_SHARED((NPAD, HH), jnp.float32),
        pltpu.VMEM_SHARED((NPAD, HH), jnp.float32),
        pltpu.VMEM_SHARED((NPAD,), jnp.float32),
        pltpu.SemaphoreType.DMA,
        pltpu.SemaphoreType.DMA,
        pltpu.SemaphoreType.DMA,
        pltpu.SemaphoreType.DMA,
        pltpu.SemaphoreType.DMA,
        pltpu.SemaphoreType.DMA,
        pltpu.SemaphoreType.DMA,
        pltpu.SemaphoreType.DMA,
        pltpu.SemaphoreType.DMA,
    ],
)(_sc_body)


def _fin_body(x_ref, a0_ref, a1_ref, d_ref, g_ref, beta_ref, o_ref):
    agg = jnp.concatenate([a0_ref[0], a1_ref[0]], axis=1)
    deg = d_ref[...]
    msg = agg / jnp.where(deg == 0.0, 1.0, deg)
    h = x_ref[...] + msg
    rms = jnp.sqrt(jnp.mean(h * h, axis=1, keepdims=True) + EPS)
    o_ref[...] = (h / rms) * g_ref[...] + beta_ref[...]


def _finalize(x, a0, a1, d, g2, beta2):
    blk = 1000
    return pl.pallas_call(
        _fin_body,
        grid=(N // blk,),
        in_specs=[
            pl.BlockSpec((blk, H), lambda i: (i, 0)),
            pl.BlockSpec((1, blk, HH), lambda i: (0, i, 0)),
            pl.BlockSpec((1, blk, HH), lambda i: (1, i, 0)),
            pl.BlockSpec((blk, 1), lambda i: (i, 0)),
            pl.BlockSpec((1, H), lambda i: (0, 0)),
            pl.BlockSpec((1, H), lambda i: (0, 0)),
        ],
        out_specs=pl.BlockSpec((blk, H), lambda i: (i, 0)),
        out_shape=jax.ShapeDtypeStruct((N, H), jnp.float32),
    )(x, a0, a1, d, g2, beta2)


def kernel(x, edge_index, W, b, g, beta):
    m2 = _linear_relu(x, W.T, b.reshape(1, H))

    npad_e = EPAD - E
    pad2 = jnp.stack([jnp.full((npad_e,), N, dtype=jnp.int32),
                      jnp.zeros((npad_e,), dtype=jnp.int32)])
    ei_p = jnp.concatenate([edge_index, pad2], axis=1)

    zacc = jnp.zeros((ROWS_PER_SUB, HH), dtype=jnp.float32)
    zdeg = jnp.zeros((ROWS_PER_SUB,), dtype=jnp.float32)
    ones = jnp.ones((CHUNK,), dtype=jnp.float32)

    agg2, deg = _sc_aggregate(m2, ei_p, zacc, zdeg, ones)

    d = deg.reshape(NC, NPAD).sum(axis=0).reshape(NPAD, 1)

    return _finalize(x, agg2, agg2, d, g.reshape(1, H), beta.reshape(1, H))


# speedup vs baseline: 1.0748x; 1.0748x over previous
import functools

import jax
import jax.numpy as jnp
from jax import lax
from jax.experimental import pallas as pl
from jax.experimental.pallas import tpu as pltpu
from jax.experimental.pallas import tpu_sc as plsc

N = 10000
E = 320000
H = 128
HH = H // 2
EPS = 1e-6

NC = 2
NS = 16
CHUNK = 64
NBUF = 4
NPAD = 10240
ROWS_PER_SUB = NPAD // NS
CPW = 320
STG = CPW // 8
NCH = NS * CPW
EPAD = NCH * CHUNK
MSTAGE = 624


def _mm_body(x_ref, wt_ref, b_ref, o_ref):
    acc = jnp.dot(x_ref[...], wt_ref[...], preferred_element_type=jnp.float32)
    m = jnp.maximum(acc + b_ref[...], 0.0)
    o_ref[0] = m[:, :HH]
    o_ref[1] = m[:, HH:]


def _linear_relu(x, wt, b2):
    blk = 1000
    return pl.pallas_call(
        _mm_body,
        grid=(N // blk,),
        in_specs=[
            pl.BlockSpec((blk, H), lambda i: (i, 0)),
            pl.BlockSpec((H, H), lambda i: (0, 0)),
            pl.BlockSpec((1, H), lambda i: (0, 0)),
        ],
        out_specs=pl.BlockSpec((2, blk, HH), lambda i: (0, i, 0)),
        out_shape=jax.ShapeDtypeStruct((2, N, HH), jnp.float32),
    )(x, wt, b2)


def _sc_body(m_hbm, ei_hbm, zacc_hbm, zdeg_hbm, ones_hbm,
             agg_out, deg_out,
             row_v, col_v, b0, b1, b2, b3, ones_v, m_sp, acc_s, deg_s,
             g0, g1, g2, g3, s0, s1, s2, s3, sem_d):
    c = lax.axis_index("c")
    s = lax.axis_index("s")

    pltpu.sync_copy(ones_hbm, ones_v)

    pltpu.sync_copy(m_hbm.at[c, pl.ds(s * MSTAGE, MSTAGE)],
                    m_sp.at[pl.ds(s * MSTAGE, MSTAGE)])

    @pl.when(s == NS - 1)
    def _():
        pltpu.sync_copy(m_hbm.at[c, pl.ds(NS * MSTAGE, N - NS * MSTAGE)],
                        m_sp.at[pl.ds(NS * MSTAGE, N - NS * MSTAGE)])

    r0 = s * ROWS_PER_SUB
    pltpu.sync_copy(zacc_hbm, acc_s.at[pl.ds(r0, ROWS_PER_SUB)])
    pltpu.sync_copy(zdeg_hbm, deg_s.at[pl.ds(r0, ROWS_PER_SUB)])
    plsc.subcore_barrier()

    bufs = (b0, b1, b2, b3)
    gsems = (g0, g1, g2, g3)
    ssems = (s0, s1, s2, s3)

    def gather_issue(lc, t):
        pltpu.async_copy(m_sp.at[col_v.at[pl.ds(lc * CHUNK, CHUNK)]], bufs[t], gsems[t])

    def gather_wait(lc, t):
        pltpu.make_async_copy(m_sp.at[col_v.at[pl.ds(lc * CHUNK, CHUNK)]], bufs[t], gsems[t]).wait()


    def scatter_wait(lc, t):
        pltpu.make_async_copy(bufs[t], acc_s.at[row_v.at[pl.ds(lc * CHUNK, CHUNK)]],
                              ssems[t]).wait()

    for q in range(8):
        deg_core = 0 if q < 4 else 1
        base = (s * CPW + q * STG) * CHUNK
        pltpu.sync_copy(ei_hbm.at[0, pl.ds(base, STG * CHUNK)], row_v)
        pltpu.sync_copy(ei_hbm.at[1, pl.ds(base, STG * CHUNK)], col_v)

        gather_issue(0, 0)
        gather_issue(1, 1)

        def body(j, carry):
            for t in range(NBUF):
                lc = j * NBUF + t
                gather_wait(lc, t)
                pltpu.async_copy(bufs[t], acc_s.at[row_v.at[pl.ds(lc * CHUNK, CHUNK)]],
                                 ssems[t], add=True)

                @pl.when(c == deg_core)
                def _():
                    pltpu.async_copy(ones_v, deg_s.at[row_v.at[pl.ds(lc * CHUNK, CHUNK)]],
                                     sem_d, add=True)
                tp = (t + 2) % NBUF

                @pl.when(lc + 2 < STG)
                def _():
                    @pl.when(lc >= 2)
                    def _():
                        scatter_wait(lc - 2, tp)
                    gather_issue(lc + 2, tp)
            return carry

        lax.fori_loop(0, STG // NBUF, body, 0)

        for t in range(NBUF):
            scatter_wait(STG - NBUF + t, t)

        def deg_drain(i, carry):
            pltpu.make_async_copy(ones_v, deg_s.at[row_v.at[pl.ds(i * CHUNK, CHUNK)]],
                                  sem_d).wait()
            return carry

        @pl.when(c == deg_core)
        def _():
            lax.fori_loop(0, STG, deg_drain, 0)

    plsc.subcore_barrier()
    pltpu.sync_copy(acc_s.at[pl.ds(r0, ROWS_PER_SUB)],
                    agg_out.at[c, pl.ds(r0, ROWS_PER_SUB)])

    pltpu.sync_copy(deg_s.at[pl.ds(r0, ROWS_PER_SUB)],
                    deg_out.at[pl.ds(c * NPAD + r0, ROWS_PER_SUB)])


_sc_aggregate = functools.partial(
    pl.kernel,
    out_type=(
        jax.ShapeDtypeStruct((NC, NPAD, HH), jnp.float32),
        jax.ShapeDtypeStruct((NC * NPAD,), jnp.float32),
    ),
    mesh=plsc.VectorSubcoreMesh(core_axis_name="c", subcore_axis_name="s"),
    compiler_params=pltpu.CompilerParams(use_tc_tiling_on_sc=False),
    scratch_types=[
        pltpu.VMEM((STG * CHUNK,), jnp.int32),
        pltpu.VMEM((STG * CHUNK,), jnp.int32),
        pltpu.VMEM((CHUNK, HH), jnp.float32),
        pltpu.VMEM((CHUNK, HH), jnp.float32),
        pltpu.VMEM((CHUNK, HH), jnp.float32),
        pltpu.VMEM((CHUNK, HH), jnp.float32),
        pltpu.VMEM((CHUNK,), jnp.float32),
        pltpu.VMEM_SHARED((NPAD, HH), jnp.float32),
        pltpu.VMEM_SHARED((NPAD, HH), jnp.float32),
        pltpu.VMEM_SHARED((NPAD,), jnp.float32),
        pltpu.SemaphoreType.DMA,
        pltpu.SemaphoreType.DMA,
        pltpu.SemaphoreType.DMA,
        pltpu.SemaphoreType.DMA,
        pltpu.SemaphoreType.DMA,
        pltpu.SemaphoreType.DMA,
        pltpu.SemaphoreType.DMA,
        pltpu.SemaphoreType.DMA,
        pltpu.SemaphoreType.DMA,
    ],
)(_sc_body)


def _fin_body(x_ref, a0_ref, a1_ref, d_ref, g_ref, beta_ref, o_ref):
    agg = jnp.concatenate([a0_ref[0], a1_ref[0]], axis=1)
    deg = d_ref[...]
    msg = agg / jnp.where(deg == 0.0, 1.0, deg)
    h = x_ref[...] + msg
    rms = jnp.sqrt(jnp.mean(h * h, axis=1, keepdims=True) + EPS)
    o_ref[...] = (h / rms) * g_ref[...] + beta_ref[...]


def _finalize(x, a0, a1, d, g2, beta2):
    blk = 1000
    return pl.pallas_call(
        _fin_body,
        grid=(N // blk,),
        in_specs=[
            pl.BlockSpec((blk, H), lambda i: (i, 0)),
            pl.BlockSpec((1, blk, HH), lambda i: (0, i, 0)),
            pl.BlockSpec((1, blk, HH), lambda i: (1, i, 0)),
            pl.BlockSpec((blk, 1), lambda i: (i, 0)),
            pl.BlockSpec((1, H), lambda i: (0, 0)),
            pl.BlockSpec((1, H), lambda i: (0, 0)),
        ],
        out_specs=pl.BlockSpec((blk, H), lambda i: (i, 0)),
        out_shape=jax.ShapeDtypeStruct((N, H), jnp.float32),
    )(x, a0, a1, d, g2, beta2)


def kernel(x, edge_index, W, b, g, beta):
    m2 = _linear_relu(x, W.T, b.reshape(1, H))

    npad_e = EPAD - E
    pad2 = jnp.stack([jnp.full((npad_e,), N, dtype=jnp.int32),
                      jnp.zeros((npad_e,), dtype=jnp.int32)])
    ei_p = jnp.concatenate([edge_index, pad2], axis=1)

    zacc = jnp.zeros((ROWS_PER_SUB, HH), dtype=jnp.float32)
    zdeg = jnp.zeros((ROWS_PER_SUB,), dtype=jnp.float32)
    ones = jnp.ones((CHUNK,), dtype=jnp.float32)

    agg2, deg = _sc_aggregate(m2, ei_p, zacc, zdeg, ones)

    d = deg.reshape(NC, NPAD).sum(axis=0).reshape(NPAD, 1)

    return _finalize(x, agg2, agg2, d, g.reshape(1, H), beta.reshape(1, H))


# speedup vs baseline: 1.1291x; 1.0505x over previous
import functools

import jax
import jax.numpy as jnp
from jax import lax
from jax.experimental import pallas as pl
from jax.experimental.pallas import tpu as pltpu
from jax.experimental.pallas import tpu_sc as plsc

N = 10000
E = 320000
H = 128
HH = H // 2
EPS = 1e-6

NC = 2
NS = 16
CHUNK = 64
NBUF = 4
NPAD = 10240
ROWS_PER_SUB = NPAD // NS
CPW = 320
STG = CPW // 8
NCH = NS * CPW
EPAD = NCH * CHUNK
MSTAGE = 624


def _mm_body(x_ref, wt_ref, b_ref, o_ref):
    acc = jnp.dot(x_ref[...], wt_ref[...], preferred_element_type=jnp.float32)
    o_ref[...] = jnp.maximum(acc + b_ref[...], 0.0)


def _linear_relu(x, wt, b2):
    blk = 1000
    return pl.pallas_call(
        _mm_body,
        grid=(N // blk,),
        in_specs=[
            pl.BlockSpec((blk, H), lambda i: (i, 0)),
            pl.BlockSpec((H, H), lambda i: (0, 0)),
            pl.BlockSpec((1, H), lambda i: (0, 0)),
        ],
        out_specs=pl.BlockSpec((blk, H), lambda i: (i, 0)),
        out_shape=jax.ShapeDtypeStruct((N, H), jnp.float32),
    )(x, wt, b2)


def _sc_body(m_hbm, ei_hbm, zacc_hbm, zdeg_hbm, ones_hbm,
             agg_out, deg_out,
             row_v, col_v, b0, b1, b2, b3, ones_v, m_sp, acc_s, deg_s,
             g0, g1, g2, g3, s0, s1, s2, s3, sem_d):
    c = lax.axis_index("c")
    s = lax.axis_index("s")

    pltpu.sync_copy(ones_hbm, ones_v)

    pltpu.sync_copy(
        m_hbm.at[pl.ds(s * MSTAGE, MSTAGE), pl.ds(c * HH, HH)],
        m_sp.at[pl.ds(s * MSTAGE, MSTAGE)])

    @pl.when(s == NS - 1)
    def _():
        pltpu.sync_copy(
            m_hbm.at[pl.ds(NS * MSTAGE, N - NS * MSTAGE), pl.ds(c * HH, HH)],
            m_sp.at[pl.ds(NS * MSTAGE, N - NS * MSTAGE)])

    r0 = s * ROWS_PER_SUB
    pltpu.sync_copy(zacc_hbm, acc_s.at[pl.ds(r0, ROWS_PER_SUB)])
    pltpu.sync_copy(zdeg_hbm, deg_s.at[pl.ds(r0, ROWS_PER_SUB)])
    plsc.subcore_barrier()

    bufs = (b0, b1, b2, b3)
    gsems = (g0, g1, g2, g3)
    ssems = (s0, s1, s2, s3)

    def gather_issue(lc, t):
        pltpu.async_copy(m_sp.at[col_v.at[pl.ds(lc * CHUNK, CHUNK)]], bufs[t], gsems[t])

    def gather_wait(lc, t):
        pltpu.make_async_copy(m_sp.at[col_v.at[pl.ds(lc * CHUNK, CHUNK)]], bufs[t], gsems[t]).wait()


    def scatter_wait(lc, t):
        pltpu.make_async_copy(bufs[t], acc_s.at[row_v.at[pl.ds(lc * CHUNK, CHUNK)]],
                              ssems[t]).wait()

    for q in range(8):
        deg_core = 0 if q < 4 else 1
        base = (s * CPW + q * STG) * CHUNK
        pltpu.sync_copy(ei_hbm.at[0, pl.ds(base, STG * CHUNK)], row_v)
        pltpu.sync_copy(ei_hbm.at[1, pl.ds(base, STG * CHUNK)], col_v)

        gather_issue(0, 0)
        gather_issue(1, 1)

        def body(j, carry):
            for t in range(NBUF):
                lc = j * NBUF + t
                gather_wait(lc, t)
                pltpu.async_copy(bufs[t], acc_s.at[row_v.at[pl.ds(lc * CHUNK, CHUNK)]],
                                 ssems[t], add=True)

                @pl.when(c == deg_core)
                def _():
                    pltpu.async_copy(ones_v, deg_s.at[row_v.at[pl.ds(lc * CHUNK, CHUNK)]],
                                     sem_d, add=True)
                tp = (t + 2) % NBUF

                @pl.when(lc + 2 < STG)
                def _():
                    @pl.when(lc >= 2)
                    def _():
                        scatter_wait(lc - 2, tp)
                    gather_issue(lc + 2, tp)
            return carry

        lax.fori_loop(0, STG // NBUF, body, 0)

        for t in range(NBUF):
            scatter_wait(STG - NBUF + t, t)

        def deg_drain(i, carry):
            pltpu.make_async_copy(ones_v, deg_s.at[row_v.at[pl.ds(i * CHUNK, CHUNK)]],
                                  sem_d).wait()
            return carry

        @pl.when(c == deg_core)
        def _():
            lax.fori_loop(0, STG, deg_drain, 0)

    plsc.subcore_barrier()
    pltpu.sync_copy(acc_s.at[pl.ds(r0, ROWS_PER_SUB)],
                    agg_out.at[c, pl.ds(r0, ROWS_PER_SUB)])

    pltpu.sync_copy(deg_s.at[pl.ds(r0, ROWS_PER_SUB)],
                    deg_out.at[pl.ds(c * NPAD + r0, ROWS_PER_SUB)])


_sc_aggregate = functools.partial(
    pl.kernel,
    out_type=(
        jax.ShapeDtypeStruct((NC, NPAD, HH), jnp.float32),
        jax.ShapeDtypeStruct((NC * NPAD,), jnp.float32),
    ),
    mesh=plsc.VectorSubcoreMesh(core_axis_name="c", subcore_axis_name="s"),
    compiler_params=pltpu.CompilerParams(use_tc_tiling_on_sc=False),
    scratch_types=[
        pltpu.VMEM((STG * CHUNK,), jnp.int32),
        pltpu.VMEM((STG * CHUNK,), jnp.int32),
        pltpu.VMEM((CHUNK, HH), jnp.float32),
        pltpu.VMEM((CHUNK, HH), jnp.float32),
        pltpu.VMEM((CHUNK, HH), jnp.float32),
        pltpu.VMEM((CHUNK, HH), jnp.float32),
        pltpu.VMEM((CHUNK,), jnp.float32),
        pltpu.VMEM_SHARED((NPAD, HH), jnp.float32),
        pltpu.VMEM_SHARED((NPAD, HH), jnp.float32),
        pltpu.VMEM_SHARED((NPAD,), jnp.float32),
        pltpu.SemaphoreType.DMA,
        pltpu.SemaphoreType.DMA,
        pltpu.SemaphoreType.DMA,
        pltpu.SemaphoreType.DMA,
        pltpu.SemaphoreType.DMA,
        pltpu.SemaphoreType.DMA,
        pltpu.SemaphoreType.DMA,
        pltpu.SemaphoreType.DMA,
        pltpu.SemaphoreType.DMA,
    ],
)(_sc_body)


def _fin_body(x_ref, a0_ref, a1_ref, d_ref, g_ref, beta_ref, o_ref):
    agg = jnp.concatenate([a0_ref[0], a1_ref[0]], axis=1)
    deg = d_ref[...]
    msg = agg / jnp.where(deg == 0.0, 1.0, deg)
    h = x_ref[...] + msg
    rms = jnp.sqrt(jnp.mean(h * h, axis=1, keepdims=True) + EPS)
    o_ref[...] = (h / rms) * g_ref[...] + beta_ref[...]


def _finalize(x, a0, a1, d, g2, beta2):
    blk = 1000
    return pl.pallas_call(
        _fin_body,
        grid=(N // blk,),
        in_specs=[
            pl.BlockSpec((blk, H), lambda i: (i, 0)),
            pl.BlockSpec((1, blk, HH), lambda i: (0, i, 0)),
            pl.BlockSpec((1, blk, HH), lambda i: (1, i, 0)),
            pl.BlockSpec((blk, 1), lambda i: (i, 0)),
            pl.BlockSpec((1, H), lambda i: (0, 0)),
            pl.BlockSpec((1, H), lambda i: (0, 0)),
        ],
        out_specs=pl.BlockSpec((blk, H), lambda i: (i, 0)),
        out_shape=jax.ShapeDtypeStruct((N, H), jnp.float32),
    )(x, a0, a1, d, g2, beta2)


def kernel(x, edge_index, W, b, g, beta):
    m2 = _linear_relu(x, W.T, b.reshape(1, H))

    npad_e = EPAD - E
    pad2 = jnp.stack([jnp.full((npad_e,), N, dtype=jnp.int32),
                      jnp.zeros((npad_e,), dtype=jnp.int32)])
    ei_p = jnp.concatenate([edge_index, pad2], axis=1)

    zacc = jnp.zeros((ROWS_PER_SUB, HH), dtype=jnp.float32)
    zdeg = jnp.zeros((ROWS_PER_SUB,), dtype=jnp.float32)
    ones = jnp.ones((CHUNK,), dtype=jnp.float32)

    agg2, deg = _sc_aggregate(m2, ei_p, zacc, zdeg, ones)

    d = (deg[:NPAD] + deg[NPAD:]).reshape(NPAD, 1)

    return _finalize(x, agg2, agg2, d, g.reshape(1, H), beta.reshape(1, H))


# speedup vs baseline: 1.1741x; 1.0398x over previous
import functools

import jax
import jax.numpy as jnp
from jax import lax
from jax.experimental import pallas as pl
from jax.experimental.pallas import tpu as pltpu
from jax.experimental.pallas import tpu_sc as plsc

N = 10000
E = 320000
H = 128
HH = H // 2
EPS = 1e-6

NC = 2
NS = 16
CHUNK = 64
NBUF = 4
NPAD = 10240
ROWS_PER_SUB = NPAD // NS
CPW = 320
STG = CPW // 8
NCH = NS * CPW
EPAD = NCH * CHUNK
MSTAGE = 624


def _mm_body(x_ref, wt_ref, b_ref, o_ref):
    acc = jnp.dot(x_ref[...], wt_ref[...], preferred_element_type=jnp.float32)
    o_ref[...] = jnp.maximum(acc + b_ref[...], 0.0)


def _linear_relu(x, wt, b2):
    blk = 1000
    return pl.pallas_call(
        _mm_body,
        grid=(N // blk,),
        in_specs=[
            pl.BlockSpec((blk, H), lambda i: (i, 0)),
            pl.BlockSpec((H, H), lambda i: (0, 0)),
            pl.BlockSpec((1, H), lambda i: (0, 0)),
        ],
        out_specs=pl.BlockSpec((blk, H), lambda i: (i, 0)),
        out_shape=jax.ShapeDtypeStruct((N, H), jnp.float32),
    )(x, wt, b2)


def _sc_body(m_hbm, ei_hbm, zacc_hbm, zdeg_hbm, ones_hbm,
             agg_out, deg_out,
             row_v, col_v, b0, b1, b2, b3, ones_v, m_sp, acc_s, deg_s,
             g0, g1, g2, g3, s0, s1, s2, s3, sem_d):
    c = lax.axis_index("c")
    s = lax.axis_index("s")

    pltpu.sync_copy(ones_hbm, ones_v)

    pltpu.sync_copy(
        m_hbm.at[pl.ds(s * MSTAGE, MSTAGE), pl.ds(c * HH, HH)],
        m_sp.at[pl.ds(s * MSTAGE, MSTAGE)])

    @pl.when(s == NS - 1)
    def _():
        pltpu.sync_copy(
            m_hbm.at[pl.ds(NS * MSTAGE, N - NS * MSTAGE), pl.ds(c * HH, HH)],
            m_sp.at[pl.ds(NS * MSTAGE, N - NS * MSTAGE)])

    r0 = s * ROWS_PER_SUB
    pltpu.sync_copy(zacc_hbm, acc_s.at[pl.ds(r0, ROWS_PER_SUB)])
    pltpu.sync_copy(zdeg_hbm, deg_s.at[pl.ds(r0, ROWS_PER_SUB)])
    plsc.subcore_barrier()

    bufs = (b0, b1, b2, b3)
    gsems = (g0, g1, g2, g3)
    ssems = (s0, s1, s2, s3)

    def gather_issue(lc, t):
        pltpu.async_copy(m_sp.at[col_v.at[pl.ds(lc * CHUNK, CHUNK)]], bufs[t], gsems[t])

    def gather_wait(lc, t):
        pltpu.make_async_copy(m_sp.at[col_v.at[pl.ds(lc * CHUNK, CHUNK)]], bufs[t], gsems[t]).wait()


    def scatter_wait(lc, t):
        pltpu.make_async_copy(bufs[t], acc_s.at[row_v.at[pl.ds(lc * CHUNK, CHUNK)]],
                              ssems[t]).wait()

    for q in range(8):
        deg_core = 0 if q < 4 else 1
        base = (s * CPW + q * STG) * CHUNK
        pltpu.sync_copy(ei_hbm.at[0, pl.ds(base, STG * CHUNK)], row_v)
        pltpu.sync_copy(ei_hbm.at[1, pl.ds(base, STG * CHUNK)], col_v)

        gather_issue(0, 0)
        gather_issue(1, 1)

        def body(j, carry):
            for t in range(NBUF):
                lc = j * NBUF + t
                gather_wait(lc, t)
                pltpu.async_copy(bufs[t], acc_s.at[row_v.at[pl.ds(lc * CHUNK, CHUNK)]],
                                 ssems[t], add=True)

                @pl.when(c == deg_core)
                def _():
                    pltpu.async_copy(ones_v, deg_s.at[row_v.at[pl.ds(lc * CHUNK, CHUNK)]],
                                     sem_d, add=True)
                tp = (t + 2) % NBUF

                @pl.when(lc + 2 < STG)
                def _():
                    @pl.when(lc >= 2)
                    def _():
                        scatter_wait(lc - 2, tp)
                    gather_issue(lc + 2, tp)
            return carry

        lax.fori_loop(0, STG // NBUF, body, 0)

        for t in range(NBUF):
            scatter_wait(STG - NBUF + t, t)

        def deg_drain(i, carry):
            pltpu.make_async_copy(ones_v, deg_s.at[row_v.at[pl.ds(i * CHUNK, CHUNK)]],
                                  sem_d).wait()
            return carry

        @pl.when(c == deg_core)
        def _():
            lax.fori_loop(0, STG, deg_drain, 0)

    plsc.subcore_barrier()
    pltpu.sync_copy(acc_s.at[pl.ds(r0, ROWS_PER_SUB)],
                    agg_out.at[pl.ds(r0, ROWS_PER_SUB), pl.ds(c * HH, HH)])

    pltpu.sync_copy(deg_s.at[pl.ds(r0, ROWS_PER_SUB)],
                    deg_out.at[pl.ds(c * NPAD + r0, ROWS_PER_SUB)])


_sc_aggregate = functools.partial(
    pl.kernel,
    out_type=(
        jax.ShapeDtypeStruct((NPAD, H), jnp.float32),
        jax.ShapeDtypeStruct((NC * NPAD,), jnp.float32),
    ),
    mesh=plsc.VectorSubcoreMesh(core_axis_name="c", subcore_axis_name="s"),
    compiler_params=pltpu.CompilerParams(use_tc_tiling_on_sc=False),
    scratch_types=[
        pltpu.VMEM((STG * CHUNK,), jnp.int32),
        pltpu.VMEM((STG * CHUNK,), jnp.int32),
        pltpu.VMEM((CHUNK, HH), jnp.float32),
        pltpu.VMEM((CHUNK, HH), jnp.float32),
        pltpu.VMEM((CHUNK, HH), jnp.float32),
        pltpu.VMEM((CHUNK, HH), jnp.float32),
        pltpu.VMEM((CHUNK,), jnp.float32),
        pltpu.VMEM_SHARED((NPAD, HH), jnp.float32),
        pltpu.VMEM_SHARED((NPAD, HH), jnp.float32),
        pltpu.VMEM_SHARED((NPAD,), jnp.float32),
        pltpu.SemaphoreType.DMA,
        pltpu.SemaphoreType.DMA,
        pltpu.SemaphoreType.DMA,
        pltpu.SemaphoreType.DMA,
        pltpu.SemaphoreType.DMA,
        pltpu.SemaphoreType.DMA,
        pltpu.SemaphoreType.DMA,
        pltpu.SemaphoreType.DMA,
        pltpu.SemaphoreType.DMA,
    ],
)(_sc_body)


def _fin_body(x_ref, a_ref, d_ref, g_ref, beta_ref, o_ref):
    agg = a_ref[...]
    deg = d_ref[...]
    msg = agg / jnp.where(deg == 0.0, 1.0, deg)
    h = x_ref[...] + msg
    rms = jnp.sqrt(jnp.mean(h * h, axis=1, keepdims=True) + EPS)
    o_ref[...] = (h / rms) * g_ref[...] + beta_ref[...]


def _finalize(x, a, d, g2, beta2):
    blk = 1000
    return pl.pallas_call(
        _fin_body,
        grid=(N // blk,),
        in_specs=[
            pl.BlockSpec((blk, H), lambda i: (i, 0)),
            pl.BlockSpec((blk, H), lambda i: (i, 0)),
            pl.BlockSpec((blk, 1), lambda i: (i, 0)),
            pl.BlockSpec((1, H), lambda i: (0, 0)),
            pl.BlockSpec((1, H), lambda i: (0, 0)),
        ],
        out_specs=pl.BlockSpec((blk, H), lambda i: (i, 0)),
        out_shape=jax.ShapeDtypeStruct((N, H), jnp.float32),
    )(x, a, d, g2, beta2)


def kernel(x, edge_index, W, b, g, beta):
    m2 = _linear_relu(x, W.T, b.reshape(1, H))

    npad_e = EPAD - E
    pad2 = jnp.stack([jnp.full((npad_e,), N, dtype=jnp.int32),
                      jnp.zeros((npad_e,), dtype=jnp.int32)])
    ei_p = jnp.concatenate([edge_index, pad2], axis=1)

    zacc = jnp.zeros((ROWS_PER_SUB, HH), dtype=jnp.float32)
    zdeg = jnp.zeros((ROWS_PER_SUB,), dtype=jnp.float32)
    ones = jnp.ones((CHUNK,), dtype=jnp.float32)

    agg2, deg = _sc_aggregate(m2, ei_p, zacc, zdeg, ones)

    d = (deg[:NPAD] + deg[NPAD:]).reshape(NPAD, 1)

    return _finalize(x, agg2, d, g.reshape(1, H), beta.reshape(1, H))


# speedup vs baseline: 1.2201x; 1.0392x over previous
import functools

import jax
import jax.numpy as jnp
from jax import lax
from jax.experimental import pallas as pl
from jax.experimental.pallas import tpu as pltpu
from jax.experimental.pallas import tpu_sc as plsc

N = 10000
E = 320000
H = 128
HH = H // 2
EPS = 1e-6

NC = 2
NS = 16
CHUNK = 64
NBUF = 4
NPAD = 10240
ROWS_PER_SUB = NPAD // NS
CPW = 320
STG = CPW // 4
NCH = NS * CPW
EPAD = NCH * CHUNK
MSTAGE = 624


def _mm_body(x_ref, wt_ref, b_ref, o_ref):
    acc = jnp.dot(x_ref[...], wt_ref[...], preferred_element_type=jnp.float32)
    o_ref[...] = jnp.maximum(acc + b_ref[...], 0.0)


def _linear_relu(x, wt, b2):
    blk = 1000
    return pl.pallas_call(
        _mm_body,
        grid=(N // blk,),
        in_specs=[
            pl.BlockSpec((blk, H), lambda i: (i, 0)),
            pl.BlockSpec((H, H), lambda i: (0, 0)),
            pl.BlockSpec((1, H), lambda i: (0, 0)),
        ],
        out_specs=pl.BlockSpec((blk, H), lambda i: (i, 0)),
        out_shape=jax.ShapeDtypeStruct((N, H), jnp.float32),
    )(x, wt, b2)


def _sc_body(m_hbm, ei_hbm, zacc_hbm, zdeg_hbm, ones_hbm,
             agg_out, deg_out,
             row_v, col_v, b0, b1, b2, b3, ones_v, m_sp, acc_s, deg_s,
             g0, g1, g2, g3, s0, s1, s2, s3, sem_d):
    c = lax.axis_index("c")
    s = lax.axis_index("s")

    pltpu.sync_copy(ones_hbm, ones_v)

    pltpu.sync_copy(
        m_hbm.at[pl.ds(s * MSTAGE, MSTAGE), pl.ds(c * HH, HH)],
        m_sp.at[pl.ds(s * MSTAGE, MSTAGE)])

    @pl.when(s == NS - 1)
    def _():
        pltpu.sync_copy(
            m_hbm.at[pl.ds(NS * MSTAGE, N - NS * MSTAGE), pl.ds(c * HH, HH)],
            m_sp.at[pl.ds(NS * MSTAGE, N - NS * MSTAGE)])

    r0 = s * ROWS_PER_SUB
    pltpu.sync_copy(zacc_hbm, acc_s.at[pl.ds(r0, ROWS_PER_SUB)])
    pltpu.sync_copy(zdeg_hbm, deg_s.at[pl.ds(r0, ROWS_PER_SUB)])
    plsc.subcore_barrier()

    bufs = (b0, b1, b2, b3)
    gsems = (g0, g1, g2, g3)
    ssems = (s0, s1, s2, s3)

    def gather_issue(lc, t):
        pltpu.async_copy(m_sp.at[col_v.at[pl.ds(lc * CHUNK, CHUNK)]], bufs[t], gsems[t])

    def gather_wait(lc, t):
        pltpu.make_async_copy(m_sp.at[col_v.at[pl.ds(lc * CHUNK, CHUNK)]], bufs[t], gsems[t]).wait()


    def scatter_wait(lc, t):
        pltpu.make_async_copy(bufs[t], acc_s.at[row_v.at[pl.ds(lc * CHUNK, CHUNK)]],
                              ssems[t]).wait()

    for q in range(4):
        deg_core = 0 if q < 2 else 1
        base = (s * CPW + q * STG) * CHUNK
        pltpu.sync_copy(ei_hbm.at[0, pl.ds(base, STG * CHUNK)], row_v)
        pltpu.sync_copy(ei_hbm.at[1, pl.ds(base, STG * CHUNK)], col_v)

        gather_issue(0, 0)
        gather_issue(1, 1)

        def body(j, carry):
            for t in range(NBUF):
                lc = j * NBUF + t
                gather_wait(lc, t)
                pltpu.async_copy(bufs[t], acc_s.at[row_v.at[pl.ds(lc * CHUNK, CHUNK)]],
                                 ssems[t], add=True)

                @pl.when(c == deg_core)
                def _():
                    pltpu.async_copy(ones_v, deg_s.at[row_v.at[pl.ds(lc * CHUNK, CHUNK)]],
                                     sem_d, add=True)
                tp = (t + 2) % NBUF

                @pl.when(lc + 2 < STG)
                def _():
                    @pl.when(lc >= 2)
                    def _():
                        scatter_wait(lc - 2, tp)
                    gather_issue(lc + 2, tp)
            return carry

        lax.fori_loop(0, STG // NBUF, body, 0)

        for t in range(NBUF):
            scatter_wait(STG - NBUF + t, t)

        def deg_drain(i, carry):
            pltpu.make_async_copy(ones_v, deg_s.at[row_v.at[pl.ds(i * CHUNK, CHUNK)]],
                                  sem_d).wait()
            return carry

        @pl.when(c == deg_core)
        def _():
            lax.fori_loop(0, STG, deg_drain, 0)

    plsc.subcore_barrier()
    pltpu.sync_copy(acc_s.at[pl.ds(r0, ROWS_PER_SUB)],
                    agg_out.at[pl.ds(r0, ROWS_PER_SUB), pl.ds(c * HH, HH)])

    pltpu.sync_copy(deg_s.at[pl.ds(r0, ROWS_PER_SUB)],
                    deg_out.at[pl.ds(c * NPAD + r0, ROWS_PER_SUB)])


_sc_aggregate = functools.partial(
    pl.kernel,
    out_type=(
        jax.ShapeDtypeStruct((NPAD, H), jnp.float32),
        jax.ShapeDtypeStruct((NC * NPAD,), jnp.float32),
    ),
    mesh=plsc.VectorSubcoreMesh(core_axis_name="c", subcore_axis_name="s"),
    compiler_params=pltpu.CompilerParams(use_tc_tiling_on_sc=False),
    scratch_types=[
        pltpu.VMEM((STG * CHUNK,), jnp.int32),
        pltpu.VMEM((STG * CHUNK,), jnp.int32),
        pltpu.VMEM((CHUNK, HH), jnp.float32),
        pltpu.VMEM((CHUNK, HH), jnp.float32),
        pltpu.VMEM((CHUNK, HH), jnp.float32),
        pltpu.VMEM((CHUNK, HH), jnp.float32),
        pltpu.VMEM((CHUNK,), jnp.float32),
        pltpu.VMEM_SHARED((NPAD, HH), jnp.float32),
        pltpu.VMEM_SHARED((NPAD, HH), jnp.float32),
        pltpu.VMEM_SHARED((NPAD,), jnp.float32),
        pltpu.SemaphoreType.DMA,
        pltpu.SemaphoreType.DMA,
        pltpu.SemaphoreType.DMA,
        pltpu.SemaphoreType.DMA,
        pltpu.SemaphoreType.DMA,
        pltpu.SemaphoreType.DMA,
        pltpu.SemaphoreType.DMA,
        pltpu.SemaphoreType.DMA,
        pltpu.SemaphoreType.DMA,
    ],
)(_sc_body)


def _fin_body(x_ref, a_ref, d_ref, g_ref, beta_ref, o_ref):
    agg = a_ref[...]
    deg = d_ref[...]
    msg = agg / jnp.where(deg == 0.0, 1.0, deg)
    h = x_ref[...] + msg
    rms = jnp.sqrt(jnp.mean(h * h, axis=1, keepdims=True) + EPS)
    o_ref[...] = (h / rms) * g_ref[...] + beta_ref[...]


def _finalize(x, a, d, g2, beta2):
    blk = 1000
    return pl.pallas_call(
        _fin_body,
        grid=(N // blk,),
        in_specs=[
            pl.BlockSpec((blk, H), lambda i: (i, 0)),
            pl.BlockSpec((blk, H), lambda i: (i, 0)),
            pl.BlockSpec((blk, 1), lambda i: (i, 0)),
            pl.BlockSpec((1, H), lambda i: (0, 0)),
            pl.BlockSpec((1, H), lambda i: (0, 0)),
        ],
        out_specs=pl.BlockSpec((blk, H), lambda i: (i, 0)),
        out_shape=jax.ShapeDtypeStruct((N, H), jnp.float32),
    )(x, a, d, g2, beta2)


def kernel(x, edge_index, W, b, g, beta):
    m2 = _linear_relu(x, W.T, b.reshape(1, H))

    npad_e = EPAD - E
    pad2 = jnp.stack([jnp.full((npad_e,), N, dtype=jnp.int32),
                      jnp.zeros((npad_e,), dtype=jnp.int32)])
    ei_p = jnp.concatenate([edge_index, pad2], axis=1)

    zacc = jnp.zeros((ROWS_PER_SUB, HH), dtype=jnp.float32)
    zdeg = jnp.zeros((ROWS_PER_SUB,), dtype=jnp.float32)
    ones = jnp.ones((CHUNK,), dtype=jnp.float32)

    agg2, deg = _sc_aggregate(m2, ei_p, zacc, zdeg, ones)

    d = (deg[:NPAD] + deg[NPAD:]).reshape(NPAD, 1)

    return _finalize(x, agg2, d, g.reshape(1, H), beta.reshape(1, H))
